# Initial kernel scaffold; baseline (speedup 1.0000x reference)
import jax
import jax.numpy as jnp
from jax.experimental import pallas as pl


def kernel(h, edge_index, edge_distance, weight, bias):
    raise NotImplementedError("write your pallas kernel here")


# speedup vs baseline: 7.6540x; 7.6540x over previous
import functools

import jax
import jax.numpy as jnp
from jax import lax
from jax.experimental import pallas as pl
from jax.experimental.pallas import tpu as pltpu
from jax.experimental.pallas import tpu_sc as plsc

N = 10000
E = 320000
D = 128
NUM_D = 8

_INFO = plsc.get_sparse_core_info()
NC = _INFO.num_cores
NS = _INFO.num_subcores
NW = NC * NS
EPT = E // NW
BATCH = 80
NB = EPT // BATCH
CHUNK = 25


def _fill_f32(ref, n, val):
    def body(i, _):
        ref[pl.ds(i * 16, 16)] = jnp.full((16,), val, jnp.float32)
        return 0
    lax.fori_loop(0, n // 16, body, 0)


def _sc_out_deg(src3):
    mesh = plsc.VectorSubcoreMesh(core_axis_name="c", subcore_axis_name="s")

    @functools.partial(
        pl.kernel, mesh=mesh,
        out_type=jax.ShapeDtypeStruct((NC, N), jnp.float32),
        scratch_types=[
            pltpu.VMEM((NB, BATCH), jnp.int32),
            pltpu.VMEM((BATCH,), jnp.float32),
            pltpu.VMEM((2000,), jnp.float32),
            pltpu.VMEM_SHARED((N,), jnp.float32),
        ],
    )
    def k(src_hbm, out_hbm, idx_v, ones_v, zeros_v, acc):
        c = lax.axis_index("c")
        s = lax.axis_index("s")
        wid = c * NS + s
        _fill_f32(ones_v, BATCH, 1.0)
        _fill_f32(zeros_v, 2000, 0.0)
        pltpu.sync_copy(src_hbm.at[wid], idx_v)

        @pl.when(s < 5)
        def _():
            pltpu.sync_copy(zeros_v, acc.at[pl.ds(s * 2000, 2000)])

        plsc.subcore_barrier()

        def body(b, _):
            pltpu.sync_copy(ones_v, acc.at[idx_v.at[b]], add=True)
            return 0
        lax.fori_loop(0, NB, body, 0)

        plsc.subcore_barrier()

        @pl.when(s == 0)
        def _():
            pltpu.sync_copy(acc, out_hbm.at[c])

    return k(src3)


def _tc_table(h, deg0, deg1):
    R = 1000

    def body(h_ref, d0_ref, d1_ref, out_ref):
        deg = d0_ref[...] + d1_ref[...]
        norm = jnp.where(deg > 0, lax.rsqrt(deg), 0.0)
        feat = h_ref[...] * norm
        for d in range(NUM_D):
            out_ref[d] = feat * (0.5 ** d)

    return pl.pallas_call(
        body,
        grid=(N // R,),
        in_specs=[pl.BlockSpec((R, D), lambda i: (i, 0)),
                  pl.BlockSpec((R, 1), lambda i: (i, 0)),
                  pl.BlockSpec((R, 1), lambda i: (i, 0))],
        out_specs=pl.BlockSpec((NUM_D, R, D), lambda i: (0, i, 0)),
        out_shape=jax.ShapeDtypeStruct((NUM_D, N, D), jnp.float32),
    )(h, deg0, deg1)


def _sc_edge_agg(table_flat, gid3, dst3):
    mesh = plsc.VectorSubcoreMesh(core_axis_name="c", subcore_axis_name="s")

    @functools.partial(
        pl.kernel, mesh=mesh,
        out_type=(jax.ShapeDtypeStruct((NC, N, D), jnp.float32),
                  jax.ShapeDtypeStruct((NC, N), jnp.float32)),
        scratch_types=[
            pltpu.VMEM((CHUNK, BATCH), jnp.int32),
            pltpu.VMEM((CHUNK, BATCH), jnp.int32),
            pltpu.VMEM((BATCH, D), jnp.float32),
            pltpu.VMEM((BATCH,), jnp.float32),
            pltpu.VMEM((2000,), jnp.float32),
            pltpu.VMEM_SHARED((N, D), jnp.float32),
            pltpu.VMEM_SHARED((N,), jnp.float32),
            pltpu.SemaphoreType.DMA,
        ],
    )
    def k(table_hbm, gid_hbm, dst_hbm, agg_hbm, indeg_hbm,
          gid_v, dst_v, rows_v, ones_v, zeros1, acc, acc1, sem):
        c = lax.axis_index("c")
        s = lax.axis_index("s")
        wid = c * NS + s
        _fill_f32(ones_v, BATCH, 1.0)
        _fill_f32(zeros1, 2000, 0.0)

        def zfill(i, _):
            rows_v[i // 8, pl.ds((i % 8) * 16, 16)] = jnp.zeros((16,), jnp.float32)
            return 0
        lax.fori_loop(0, BATCH * 8, zfill, 0)

        @pl.when(s < 10)
        def _():
            for j in range(12):
                pltpu.sync_copy(rows_v, acc.at[pl.ds(s * 1000 + j * 80, 80)])
            pltpu.sync_copy(rows_v.at[pl.ds(0, 40)],
                            acc.at[pl.ds(s * 1000 + 960, 40)])

        @pl.when(s < 5)
        def _():
            pltpu.sync_copy(zeros1, acc1.at[pl.ds(s * 2000, 2000)])

        plsc.subcore_barrier()

        def chunk_body(g, _):
            pltpu.sync_copy(gid_hbm.at[wid, g], gid_v)
            pltpu.sync_copy(dst_hbm.at[wid, g], dst_v)

            def body(b, _):
                pltpu.async_copy(table_hbm.at[gid_v.at[b]], rows_v, sem).wait()
                pltpu.sync_copy(rows_v, acc.at[dst_v.at[b]], add=True)
                pltpu.sync_copy(ones_v, acc1.at[dst_v.at[b]], add=True)
                return 0
            lax.fori_loop(0, CHUNK, body, 0)
            return 0
        lax.fori_loop(0, NB // CHUNK, chunk_body, 0)

        plsc.subcore_barrier()

        @pl.when(s < 10)
        def _():
            pltpu.sync_copy(acc.at[pl.ds(s * 1000, 1000)],
                            agg_hbm.at[c, pl.ds(s * 1000, 1000)])

        @pl.when(s == 10)
        def _():
            pltpu.sync_copy(acc1, indeg_hbm.at[c])

    return k(table_flat, gid3, dst3)


def _tc_final(agg_parts, ind0, ind1, table, weight, bias2):
    R = 1000

    def body(agg_ref, i0_ref, i1_ref, feat_ref, w_ref, b_ref, out_ref):
        aggp = agg_ref[...]
        a = aggp[0] + aggp[1]
        ind = i0_ref[...] + i1_ref[...]
        mean = a / jnp.maximum(ind, 1.0)
        agg = jnp.where(ind > 0, mean, feat_ref[0])
        rst = jnp.dot(agg, w_ref[...], preferred_element_type=jnp.float32)
        norm_in = jnp.where(ind > 0, lax.rsqrt(ind), 0.0)
        out_ref[...] = rst * norm_in + b_ref[...]

    return pl.pallas_call(
        body,
        grid=(N // R,),
        in_specs=[pl.BlockSpec((NC, R, D), lambda i: (0, i, 0)),
                  pl.BlockSpec((R, 1), lambda i: (i, 0)),
                  pl.BlockSpec((R, 1), lambda i: (i, 0)),
                  pl.BlockSpec((1, R, D), lambda i: (0, i, 0)),
                  pl.BlockSpec((D, D), lambda i: (0, 0)),
                  pl.BlockSpec((1, D), lambda i: (0, 0))],
        out_specs=pl.BlockSpec((R, D), lambda i: (i, 0)),
        out_shape=jax.ShapeDtypeStruct((N, D), jnp.float32),
    )(agg_parts, ind0, ind1, table, weight, bias2)


def kernel(h, edge_index, edge_distance, weight, bias):
    src = edge_index[0]
    dst = edge_index[1]
    gid = edge_distance * N + src
    src3 = src.reshape(NW, NB, BATCH)
    gid3 = gid.reshape(NW, NB // CHUNK, CHUNK, BATCH)
    dst3 = dst.reshape(NW, NB // CHUNK, CHUNK, BATCH)

    outdeg_parts = _sc_out_deg(src3)
    table = _tc_table(h, outdeg_parts[0].reshape(N, 1),
                      outdeg_parts[1].reshape(N, 1))
    table_flat = table.reshape(NUM_D * N, D)
    agg_parts, indeg_parts = _sc_edge_agg(table_flat, gid3, dst3)
    return _tc_final(agg_parts, indeg_parts[0].reshape(N, 1),
                     indeg_parts[1].reshape(N, 1), table,
                     weight, bias.reshape(1, D))
